# baseline (speedup 1.0000x reference)
import math
import jax
import jax.numpy as jnp
from jax.experimental import pallas as pl

B = 32
LT = 64
LS = 256
N = LT + 2 * LS
C = 768
H = 12
DH = C // H
HID = 3072
KEEP = 180
NKEPT = LT + 2 * KEEP


def _layernorm(x, w, b, eps=1e-5):
    mu = jnp.mean(x, axis=-1, keepdims=True)
    var = jnp.mean((x - mu) ** 2, axis=-1, keepdims=True)
    return (x - mu) / jnp.sqrt(var + eps) * w + b


def _attn_proj_body(x_ref, attn_ref, v_ref, pw_ref, pb_ref, out_ref):
    xa = jnp.concatenate(
        [jnp.dot(attn_ref[0, h], v_ref[0, h],
                 preferred_element_type=jnp.float32) for h in range(H)],
        axis=1)
    part = jnp.dot(xa, pw_ref[...], preferred_element_type=jnp.float32)
    out_ref[0] = x_ref[0] + pb_ref[0] + part


def _attn_proj(x, attn, v, proj_w, proj_b):
    pw = proj_w.T
    pb = proj_b.reshape(1, C)
    return pl.pallas_call(
        _attn_proj_body,
        grid=(B,),
        in_specs=[
            pl.BlockSpec((1, N, C), lambda b: (b, 0, 0)),
            pl.BlockSpec((1, H, N, N), lambda b: (b, 0, 0, 0)),
            pl.BlockSpec((1, H, N, DH), lambda b: (b, 0, 0, 0)),
            pl.BlockSpec((C, C), lambda b: (0, 0)),
            pl.BlockSpec((1, C), lambda b: (0, 0)),
        ],
        out_specs=pl.BlockSpec((1, N, C), lambda b: (b, 0, 0)),
        out_shape=jax.ShapeDtypeStruct((B, N, C), jnp.float32),
    )(x, attn, v, pw, pb)


def _gather_mlp_body(x2_ref, idx_ref, n2w_ref, n2b_ref, f1w_ref, f1b_ref,
                     f2w_ref, f2b_ref, out_ref):
    idx = idx_ref[0, 0, :NKEPT]
    onehot = (idx[:, None] ==
              jax.lax.broadcasted_iota(jnp.int32, (NKEPT, N), 1))
    g = jnp.dot(onehot.astype(jnp.float32), x2_ref[0],
                preferred_element_type=jnp.float32)
    mu = jnp.mean(g, axis=-1, keepdims=True)
    var = jnp.mean((g - mu) ** 2, axis=-1, keepdims=True)
    hn = (g - mu) / jnp.sqrt(var + 1e-5) * n2w_ref[0] + n2b_ref[0]
    a1 = jnp.dot(hn, f1w_ref[...],
                 preferred_element_type=jnp.float32) + f1b_ref[0]
    a1 = 0.5 * a1 * (1.0 + jax.lax.erf(a1 * (2.0 ** -0.5)))
    a2 = jnp.dot(a1, f2w_ref[...],
                 preferred_element_type=jnp.float32) + f2b_ref[0]
    out_ref[0] = g + a2


def _gather_mlp(x2, row_idx, n2w, n2b, f1w, f1b, f2w, f2b):
    idx_pad = jnp.pad(row_idx, ((0, 0), (0, 512 - NKEPT))).reshape(B, 1, 512)
    return pl.pallas_call(
        _gather_mlp_body,
        grid=(B,),
        in_specs=[
            pl.BlockSpec((1, N, C), lambda b: (b, 0, 0)),
            pl.BlockSpec((1, 1, 512), lambda b: (b, 0, 0)),
            pl.BlockSpec((1, C), lambda b: (0, 0)),
            pl.BlockSpec((1, C), lambda b: (0, 0)),
            pl.BlockSpec((C, HID), lambda b: (0, 0)),
            pl.BlockSpec((1, HID), lambda b: (0, 0)),
            pl.BlockSpec((HID, C), lambda b: (0, 0)),
            pl.BlockSpec((1, C), lambda b: (0, 0)),
        ],
        out_specs=pl.BlockSpec((1, NKEPT, C), lambda b: (b, 0, 0)),
        out_shape=jax.ShapeDtypeStruct((B, NKEPT, C), jnp.float32),
    )(x2, idx_pad, n2w.reshape(1, C), n2b.reshape(1, C),
      f1w.T, f1b.reshape(1, HID),
      f2w.T, f2b.reshape(1, C))


def kernel(x, global_index_template, global_index_ps, global_index_search,
           norm1_w, norm1_b, qkv_w, qkv_b, proj_w, proj_b,
           norm2_w, norm2_b, fc1_w, fc1_b, fc2_w, fc2_b):
    scale = DH ** -0.5

    h = _layernorm(x, norm1_w, norm1_b)
    qkv = h @ qkv_w.T + qkv_b
    qkv = qkv.reshape(B, N, 3, H, DH).transpose(2, 0, 3, 1, 4)
    q, k, v = qkv[0], qkv[1], qkv[2]
    attn = jnp.einsum('bhqd,bhkd->bhqk', q, k) * scale
    attn = jax.nn.softmax(attn, axis=-1)

    attn_t = attn[:, :, :LT, LT:]
    attn_t = attn_t.mean(axis=2).mean(axis=1)
    attn_t_ps = attn_t[:, :LS]
    attn_t_s = attn_t[:, LS:]
    idx_ps = jnp.argsort(-attn_t_ps, axis=1)
    idx_s = jnp.argsort(-attn_t_s, axis=1)
    topk_idx_ps = idx_ps[:, :KEEP]
    topk_idx_s = idx_s[:, :KEEP]
    keep_index_ps = jnp.take_along_axis(global_index_ps, topk_idx_ps, axis=1)
    removed_index_ps = jnp.take_along_axis(global_index_ps, idx_ps[:, KEEP:], axis=1)
    keep_index_s = jnp.take_along_axis(global_index_search, topk_idx_s, axis=1)
    removed_index_s = jnp.take_along_axis(global_index_search, idx_s[:, KEEP:], axis=1)

    x2 = _attn_proj(x, attn, v, proj_w, proj_b)

    row_idx = jnp.concatenate(
        [jnp.broadcast_to(jnp.arange(LT, dtype=jnp.int32), (B, LT)),
         topk_idx_ps + LT, topk_idx_s + LT + LS], axis=1)
    x_out = _gather_mlp(x2, row_idx, norm2_w, norm2_b,
                        fc1_w, fc1_b, fc2_w, fc2_b)

    return (x_out, global_index_template, keep_index_ps, keep_index_s,
            removed_index_ps, removed_index_s, attn)


# speedup vs baseline: 1.0723x; 1.0723x over previous
import math
import jax
import jax.numpy as jnp
from jax.experimental import pallas as pl

B = 32
LT = 64
LS = 256
N = LT + 2 * LS
C = 768
H = 12
DH = C // H
HID = 3072
KEEP = 180
NKEPT = LT + 2 * KEEP


def _layernorm(x, w, b, eps=1e-5):
    mu = jnp.mean(x, axis=-1, keepdims=True)
    var = jnp.mean((x - mu) ** 2, axis=-1, keepdims=True)
    return (x - mu) / jnp.sqrt(var + eps) * w + b


def _attn_proj_body(x_ref, qkv_ref, pw_ref, pb_ref, out_ref):
    qkv2d = qkv_ref[0]
    scale = DH ** -0.5
    xa_parts = []
    for h in range(H):
        qh = qkv2d[:, h * DH:(h + 1) * DH]
        kh = qkv2d[:, C + h * DH:C + (h + 1) * DH]
        vh = qkv2d[:, 2 * C + h * DH:2 * C + (h + 1) * DH]
        s = jax.lax.dot_general(qh, kh, (((1,), (1,)), ((), ())),
                                preferred_element_type=jnp.float32) * scale
        m = jnp.max(s, axis=1, keepdims=True)
        e = jnp.exp(s - m)
        denom = jnp.sum(e, axis=1, keepdims=True)
        p = e / denom
        xa_parts.append(jnp.dot(p, vh, preferred_element_type=jnp.float32))
    xa = jnp.concatenate(xa_parts, axis=1)
    part = jnp.dot(xa, pw_ref[...], preferred_element_type=jnp.float32)
    out_ref[0] = x_ref[0] + pb_ref[0] + part


def _attn_proj(x, qkv, proj_w, proj_b):
    pw = proj_w.T
    pb = proj_b.reshape(1, C)
    return pl.pallas_call(
        _attn_proj_body,
        grid=(B,),
        in_specs=[
            pl.BlockSpec((1, N, C), lambda b: (b, 0, 0)),
            pl.BlockSpec((1, N, 3 * C), lambda b: (b, 0, 0)),
            pl.BlockSpec((C, C), lambda b: (0, 0)),
            pl.BlockSpec((1, C), lambda b: (0, 0)),
        ],
        out_specs=pl.BlockSpec((1, N, C), lambda b: (b, 0, 0)),
        out_shape=jax.ShapeDtypeStruct((B, N, C), jnp.float32),
    )(x, qkv, pw, pb)


def _gather_mlp_body(x2_ref, idx_ref, n2w_ref, n2b_ref, f1w_ref, f1b_ref,
                     f2w_ref, f2b_ref, out_ref):
    idx = idx_ref[0, 0, :NKEPT]
    onehot = (idx[:, None] ==
              jax.lax.broadcasted_iota(jnp.int32, (NKEPT, N), 1))
    g = jnp.dot(onehot.astype(jnp.float32), x2_ref[0],
                preferred_element_type=jnp.float32)
    mu = jnp.mean(g, axis=-1, keepdims=True)
    var = jnp.mean((g - mu) ** 2, axis=-1, keepdims=True)
    hn = (g - mu) / jnp.sqrt(var + 1e-5) * n2w_ref[0] + n2b_ref[0]
    a1 = jnp.dot(hn, f1w_ref[...],
                 preferred_element_type=jnp.float32) + f1b_ref[0]
    a1 = 0.5 * a1 * (1.0 + jax.lax.erf(a1 * (2.0 ** -0.5)))
    a2 = jnp.dot(a1, f2w_ref[...],
                 preferred_element_type=jnp.float32) + f2b_ref[0]
    out_ref[0] = g + a2


def _gather_mlp(x2, row_idx, n2w, n2b, f1w, f1b, f2w, f2b):
    idx_pad = jnp.pad(row_idx, ((0, 0), (0, 512 - NKEPT))).reshape(B, 1, 512)
    return pl.pallas_call(
        _gather_mlp_body,
        grid=(B,),
        in_specs=[
            pl.BlockSpec((1, N, C), lambda b: (b, 0, 0)),
            pl.BlockSpec((1, 1, 512), lambda b: (b, 0, 0)),
            pl.BlockSpec((1, C), lambda b: (0, 0)),
            pl.BlockSpec((1, C), lambda b: (0, 0)),
            pl.BlockSpec((C, HID), lambda b: (0, 0)),
            pl.BlockSpec((1, HID), lambda b: (0, 0)),
            pl.BlockSpec((HID, C), lambda b: (0, 0)),
            pl.BlockSpec((1, C), lambda b: (0, 0)),
        ],
        out_specs=pl.BlockSpec((1, NKEPT, C), lambda b: (b, 0, 0)),
        out_shape=jax.ShapeDtypeStruct((B, NKEPT, C), jnp.float32),
    )(x2, idx_pad, n2w.reshape(1, C), n2b.reshape(1, C),
      f1w.T, f1b.reshape(1, HID),
      f2w.T, f2b.reshape(1, C))


def kernel(x, global_index_template, global_index_ps, global_index_search,
           norm1_w, norm1_b, qkv_w, qkv_b, proj_w, proj_b,
           norm2_w, norm2_b, fc1_w, fc1_b, fc2_w, fc2_b):
    scale = DH ** -0.5

    h = _layernorm(x, norm1_w, norm1_b)
    qkv = h @ qkv_w.T + qkv_b

    qkv_r = qkv.reshape(B, N, 3, H, DH).transpose(2, 0, 3, 1, 4)
    q, k = qkv_r[0], qkv_r[1]
    attn = jnp.einsum('bhqd,bhkd->bhqk', q, k) * scale
    attn = jax.nn.softmax(attn, axis=-1)

    x2 = _attn_proj(x, qkv, proj_w, proj_b)

    attn_t = attn[:, :, :LT, LT:]
    attn_t = attn_t.mean(axis=2).mean(axis=1)

    attn_t_ps = attn_t[:, :LS]
    attn_t_s = attn_t[:, LS:]
    idx_ps = jnp.argsort(-attn_t_ps, axis=1)
    idx_s = jnp.argsort(-attn_t_s, axis=1)
    topk_idx_ps = idx_ps[:, :KEEP]
    topk_idx_s = idx_s[:, :KEEP]
    keep_index_ps = jnp.take_along_axis(global_index_ps, topk_idx_ps, axis=1)
    removed_index_ps = jnp.take_along_axis(global_index_ps, idx_ps[:, KEEP:], axis=1)
    keep_index_s = jnp.take_along_axis(global_index_search, topk_idx_s, axis=1)
    removed_index_s = jnp.take_along_axis(global_index_search, idx_s[:, KEEP:], axis=1)

    row_idx = jnp.concatenate(
        [jnp.broadcast_to(jnp.arange(LT, dtype=jnp.int32), (B, LT)),
         topk_idx_ps + LT, topk_idx_s + LT + LS], axis=1)
    x_out = _gather_mlp(x2, row_idx, norm2_w, norm2_b,
                        fc1_w, fc1_b, fc2_w, fc2_b)

    return (x_out, global_index_template, keep_index_ps, keep_index_s,
            removed_index_ps, removed_index_s, attn)


# speedup vs baseline: 1.2951x; 1.2078x over previous
import math
import jax
import jax.numpy as jnp
from jax.experimental import pallas as pl

B = 32
LT = 64
LS = 256
N = LT + 2 * LS
C = 768
H = 12
DH = C // H
HID = 3072
KEEP = 180
NKEPT = LT + 2 * KEEP


def _layernorm(x, w, b, eps=1e-5):
    mu = jnp.mean(x, axis=-1, keepdims=True)
    var = jnp.mean((x - mu) ** 2, axis=-1, keepdims=True)
    return (x - mu) / jnp.sqrt(var + eps) * w + b


def _attn_proj_body(x_ref, qkv_ref, pw_ref, pb_ref, s_ref, out_ref):
    qkv2d = qkv_ref[0]
    scale = DH ** -0.5
    xa_parts = []
    for h in range(H):
        qh = qkv2d[:, h * DH:(h + 1) * DH]
        kh = qkv2d[:, C + h * DH:C + (h + 1) * DH]
        vh = qkv2d[:, 2 * C + h * DH:2 * C + (h + 1) * DH]
        s0 = jax.lax.dot_general(qh, kh, (((1,), (1,)), ((), ())),
                                 preferred_element_type=jnp.float32)
        s_ref[0, h] = s0
        s = s0 * scale
        m = jnp.max(s, axis=1, keepdims=True)
        e = jnp.exp(s - m)
        denom = jnp.sum(e, axis=1, keepdims=True)
        p = e / denom
        xa_parts.append(jnp.dot(p, vh, preferred_element_type=jnp.float32))
    xa = jnp.concatenate(xa_parts, axis=1)
    part = jnp.dot(xa, pw_ref[...], preferred_element_type=jnp.float32)
    out_ref[0] = x_ref[0] + pb_ref[0] + part


def _attn_proj(x, qkv, proj_w, proj_b):
    pw = proj_w.T
    pb = proj_b.reshape(1, C)
    return pl.pallas_call(
        _attn_proj_body,
        grid=(B,),
        in_specs=[
            pl.BlockSpec((1, N, C), lambda b: (b, 0, 0)),
            pl.BlockSpec((1, N, 3 * C), lambda b: (b, 0, 0)),
            pl.BlockSpec((C, C), lambda b: (0, 0)),
            pl.BlockSpec((1, C), lambda b: (0, 0)),
        ],
        out_specs=[
            pl.BlockSpec((1, H, N, N), lambda b: (b, 0, 0, 0)),
            pl.BlockSpec((1, N, C), lambda b: (b, 0, 0)),
        ],
        out_shape=[
            jax.ShapeDtypeStruct((B, H, N, N), jnp.float32),
            jax.ShapeDtypeStruct((B, N, C), jnp.float32),
        ],
    )(x, qkv, pw, pb)


def _gather_mlp_body(x2_ref, idx_ref, n2w_ref, n2b_ref, f1w_ref, f1b_ref,
                     f2w_ref, f2b_ref, out_ref):
    idx = idx_ref[0, 0, :NKEPT]
    onehot = (idx[:, None] ==
              jax.lax.broadcasted_iota(jnp.int32, (NKEPT, N), 1))
    g = jnp.dot(onehot.astype(jnp.float32), x2_ref[0],
                preferred_element_type=jnp.float32)
    mu = jnp.mean(g, axis=-1, keepdims=True)
    var = jnp.mean((g - mu) ** 2, axis=-1, keepdims=True)
    hn = (g - mu) / jnp.sqrt(var + 1e-5) * n2w_ref[0] + n2b_ref[0]
    a1 = jnp.dot(hn, f1w_ref[...],
                 preferred_element_type=jnp.float32) + f1b_ref[0]
    a1 = 0.5 * a1 * (1.0 + jax.lax.erf(a1 * (2.0 ** -0.5)))
    a2 = jnp.dot(a1, f2w_ref[...],
                 preferred_element_type=jnp.float32) + f2b_ref[0]
    out_ref[0] = g + a2


def _gather_mlp(x2, row_idx, n2w, n2b, f1w, f1b, f2w, f2b):
    idx_pad = jnp.pad(row_idx, ((0, 0), (0, 512 - NKEPT))).reshape(B, 1, 512)
    return pl.pallas_call(
        _gather_mlp_body,
        grid=(B,),
        in_specs=[
            pl.BlockSpec((1, N, C), lambda b: (b, 0, 0)),
            pl.BlockSpec((1, 1, 512), lambda b: (b, 0, 0)),
            pl.BlockSpec((1, C), lambda b: (0, 0)),
            pl.BlockSpec((1, C), lambda b: (0, 0)),
            pl.BlockSpec((C, HID), lambda b: (0, 0)),
            pl.BlockSpec((1, HID), lambda b: (0, 0)),
            pl.BlockSpec((HID, C), lambda b: (0, 0)),
            pl.BlockSpec((1, C), lambda b: (0, 0)),
        ],
        out_specs=pl.BlockSpec((1, NKEPT, C), lambda b: (b, 0, 0)),
        out_shape=jax.ShapeDtypeStruct((B, NKEPT, C), jnp.float32),
    )(x2, idx_pad, n2w.reshape(1, C), n2b.reshape(1, C),
      f1w.T, f1b.reshape(1, HID),
      f2w.T, f2b.reshape(1, C))


def kernel(x, global_index_template, global_index_ps, global_index_search,
           norm1_w, norm1_b, qkv_w, qkv_b, proj_w, proj_b,
           norm2_w, norm2_b, fc1_w, fc1_b, fc2_w, fc2_b):
    scale = DH ** -0.5

    h = _layernorm(x, norm1_w, norm1_b)
    qkv = h @ qkv_w.T + qkv_b

    s_raw, x2 = _attn_proj(x, qkv, proj_w, proj_b)
    attn = jax.nn.softmax(s_raw * scale, axis=-1)

    attn_t = attn[:, :, :LT, LT:]
    attn_t = attn_t.mean(axis=2).mean(axis=1)

    attn_t_ps = attn_t[:, :LS]
    attn_t_s = attn_t[:, LS:]
    idx_ps = jnp.argsort(-attn_t_ps, axis=1)
    idx_s = jnp.argsort(-attn_t_s, axis=1)
    topk_idx_ps = idx_ps[:, :KEEP]
    topk_idx_s = idx_s[:, :KEEP]
    keep_index_ps = jnp.take_along_axis(global_index_ps, topk_idx_ps, axis=1)
    removed_index_ps = jnp.take_along_axis(global_index_ps, idx_ps[:, KEEP:], axis=1)
    keep_index_s = jnp.take_along_axis(global_index_search, topk_idx_s, axis=1)
    removed_index_s = jnp.take_along_axis(global_index_search, idx_s[:, KEEP:], axis=1)

    row_idx = jnp.concatenate(
        [jnp.broadcast_to(jnp.arange(LT, dtype=jnp.int32), (B, LT)),
         topk_idx_ps + LT, topk_idx_s + LT + LS], axis=1)
    x_out = _gather_mlp(x2, row_idx, norm2_w, norm2_b,
                        fc1_w, fc1_b, fc2_w, fc2_b)

    return (x_out, global_index_template, keep_index_ps, keep_index_s,
            removed_index_ps, removed_index_s, attn)


# speedup vs baseline: 2.5551x; 1.9729x over previous
import math
import jax
import jax.numpy as jnp
from jax.experimental import pallas as pl
from jax.experimental.pallas import tpu as pltpu

B = 32
LT = 64
LS = 256
N = LT + 2 * LS
C = 768
H = 12
DH = C // H
HID = 3072
KEEP = 180
NKEPT = LT + 2 * KEEP


def _layernorm(x, w, b, eps=1e-5):
    mu = jnp.mean(x, axis=-1, keepdims=True)
    var = jnp.mean((x - mu) ** 2, axis=-1, keepdims=True)
    return (x - mu) / jnp.sqrt(var + eps) * w + b


def _attn_proj_body(x_ref, qkv_ref, pw_ref, pb_ref, st_ref, attn_ref, out_ref):
    qkv2d = qkv_ref[0]
    scale = DH ** -0.5
    xa_parts = []
    for h in range(H):
        qh = qkv2d[:, h * DH:(h + 1) * DH]
        kh = qkv2d[:, C + h * DH:C + (h + 1) * DH]
        vh = qkv2d[:, 2 * C + h * DH:2 * C + (h + 1) * DH]
        s0 = jax.lax.dot_general(qh, kh, (((1,), (1,)), ((), ())),
                                 preferred_element_type=jnp.float32)
        st_ref[0, h] = s0[:LT]
        s = s0 * scale
        m = jnp.max(s, axis=1, keepdims=True)
        e = jnp.exp(s - m)
        denom = jnp.sum(e, axis=1, keepdims=True)
        p = e / denom
        attn_ref[0, h] = p
        xa_parts.append(jnp.dot(p, vh, preferred_element_type=jnp.float32))
    xa = jnp.concatenate(xa_parts, axis=1)
    part = jnp.dot(xa, pw_ref[...], preferred_element_type=jnp.float32)
    out_ref[0] = x_ref[0] + pb_ref[0] + part


def _attn_proj(x, qkv, proj_w, proj_b):
    pw = proj_w.T
    pb = proj_b.reshape(1, C)
    return pl.pallas_call(
        _attn_proj_body,
        grid=(B,),
        in_specs=[
            pl.BlockSpec((1, N, C), lambda b: (b, 0, 0)),
            pl.BlockSpec((1, N, 3 * C), lambda b: (b, 0, 0)),
            pl.BlockSpec((C, C), lambda b: (0, 0)),
            pl.BlockSpec((1, C), lambda b: (0, 0)),
        ],
        out_specs=[
            pl.BlockSpec((1, H, LT, N), lambda b: (b, 0, 0, 0)),
            pl.BlockSpec((1, H, N, N), lambda b: (b, 0, 0, 0)),
            pl.BlockSpec((1, N, C), lambda b: (b, 0, 0)),
        ],
        out_shape=[
            jax.ShapeDtypeStruct((B, H, LT, N), jnp.float32),
            jax.ShapeDtypeStruct((B, H, N, N), jnp.float32),
            jax.ShapeDtypeStruct((B, N, C), jnp.float32),
        ],
        compiler_params=pltpu.CompilerParams(
            vmem_limit_bytes=100 * 1024 * 1024),
    )(x, qkv, pw, pb)


def _gather_mlp_body(x2_ref, idx_ref, n2w_ref, n2b_ref, f1w_ref, f1b_ref,
                     f2w_ref, f2b_ref, out_ref):
    idx = idx_ref[0, 0, :NKEPT]
    onehot = (idx[:, None] ==
              jax.lax.broadcasted_iota(jnp.int32, (NKEPT, N), 1))
    g = jnp.dot(onehot.astype(jnp.float32), x2_ref[0],
                preferred_element_type=jnp.float32)
    mu = jnp.mean(g, axis=-1, keepdims=True)
    var = jnp.mean((g - mu) ** 2, axis=-1, keepdims=True)
    hn = (g - mu) / jnp.sqrt(var + 1e-5) * n2w_ref[0] + n2b_ref[0]
    a1 = jnp.dot(hn, f1w_ref[...],
                 preferred_element_type=jnp.float32) + f1b_ref[0]
    a1 = 0.5 * a1 * (1.0 + jax.lax.erf(a1 * (2.0 ** -0.5)))
    a2 = jnp.dot(a1, f2w_ref[...],
                 preferred_element_type=jnp.float32) + f2b_ref[0]
    out_ref[0] = g + a2


def _gather_mlp(x2, row_idx, n2w, n2b, f1w, f1b, f2w, f2b):
    idx_pad = jnp.pad(row_idx, ((0, 0), (0, 512 - NKEPT))).reshape(B, 1, 512)
    return pl.pallas_call(
        _gather_mlp_body,
        grid=(B,),
        in_specs=[
            pl.BlockSpec((1, N, C), lambda b: (b, 0, 0)),
            pl.BlockSpec((1, 1, 512), lambda b: (b, 0, 0)),
            pl.BlockSpec((1, C), lambda b: (0, 0)),
            pl.BlockSpec((1, C), lambda b: (0, 0)),
            pl.BlockSpec((C, HID), lambda b: (0, 0)),
            pl.BlockSpec((1, HID), lambda b: (0, 0)),
            pl.BlockSpec((HID, C), lambda b: (0, 0)),
            pl.BlockSpec((1, C), lambda b: (0, 0)),
        ],
        out_specs=pl.BlockSpec((1, NKEPT, C), lambda b: (b, 0, 0)),
        out_shape=jax.ShapeDtypeStruct((B, NKEPT, C), jnp.float32),
    )(x2, idx_pad, n2w.reshape(1, C), n2b.reshape(1, C),
      f1w.T, f1b.reshape(1, HID),
      f2w.T, f2b.reshape(1, C))


def kernel(x, global_index_template, global_index_ps, global_index_search,
           norm1_w, norm1_b, qkv_w, qkv_b, proj_w, proj_b,
           norm2_w, norm2_b, fc1_w, fc1_b, fc2_w, fc2_b):
    scale = DH ** -0.5

    h = _layernorm(x, norm1_w, norm1_b)
    qkv = h @ qkv_w.T + qkv_b

    s_t, attn, x2 = _attn_proj(x, qkv, proj_w, proj_b)
    p_t = jax.nn.softmax(s_t * scale, axis=-1)
    p_t = jax.lax.optimization_barrier(p_t)
    attn_t = p_t[:, :, :, LT:].mean(axis=2).mean(axis=1)

    attn_t_ps = attn_t[:, :LS]
    attn_t_s = attn_t[:, LS:]
    idx_ps = jnp.argsort(-attn_t_ps, axis=1)
    idx_s = jnp.argsort(-attn_t_s, axis=1)
    topk_idx_ps = idx_ps[:, :KEEP]
    topk_idx_s = idx_s[:, :KEEP]
    keep_index_ps = jnp.take_along_axis(global_index_ps, topk_idx_ps, axis=1)
    removed_index_ps = jnp.take_along_axis(global_index_ps, idx_ps[:, KEEP:], axis=1)
    keep_index_s = jnp.take_along_axis(global_index_search, topk_idx_s, axis=1)
    removed_index_s = jnp.take_along_axis(global_index_search, idx_s[:, KEEP:], axis=1)

    row_idx = jnp.concatenate(
        [jnp.broadcast_to(jnp.arange(LT, dtype=jnp.int32), (B, LT)),
         topk_idx_ps + LT, topk_idx_s + LT + LS], axis=1)
    x_out = _gather_mlp(x2, row_idx, norm2_w, norm2_b,
                        fc1_w, fc1_b, fc2_w, fc2_b)

    return (x_out, global_index_template, keep_index_ps, keep_index_s,
            removed_index_ps, removed_index_s, attn)
